# Initial kernel scaffold; baseline (speedup 1.0000x reference)
import jax
import jax.numpy as jnp
from jax.experimental import pallas as pl


def kernel(input):
    raise NotImplementedError("write your pallas kernel here")


# speedup vs baseline: 1.3268x; 1.3268x over previous
import functools

import jax
import jax.numpy as jnp
from jax.experimental import pallas as pl
from jax.experimental.pallas import tpu as pltpu

B, C, H, W = 4, 192, 224, 224
CB = 16
NCB = C // CB
K = 8
BIG = 1 << 30


def _copy_mag_kernel(x_ref, y_ref, idx_ref, mag_ref):
    b = pl.program_id(0)
    cb = pl.program_id(1)
    x = x_ref[...]
    y_ref[...] = x
    mags = jnp.sum(x * x, axis=(-1, -2))
    mag_ref[pl.ds(cb, 1), :] = mags

    @pl.when(cb == NCB - 1)
    def _finish_batch():
        m = mag_ref[...]
        minval = jnp.min(m)
        r = jax.lax.broadcasted_iota(jnp.int32, (NCB, CB), 0)
        c = jax.lax.broadcasted_iota(jnp.int32, (NCB, CB), 1)
        flat = r * CB + c
        cand = jnp.where(m <= minval, flat, BIG)
        first = jnp.min(cand)
        lane = jax.lax.broadcasted_iota(jnp.int32, (1, 128), 1)
        row = jnp.full((1, 128), first, dtype=jnp.int32)
        cur = cand
        for k in range(K):
            mk = jnp.min(cur)
            sel = jnp.where(mk >= BIG, first, mk)
            row = jnp.where(lane == k, sel, row)
            cur = jnp.where(cand == mk, BIG, cur)
        idx_ref[pl.ds(b, 1), :] = row


def _zero_kernel(idx_ref, y_in_ref, y_out_ref):
    del idx_ref, y_in_ref
    y_out_ref[...] = jnp.zeros_like(y_out_ref)


@jax.jit
def kernel(input):
    y1, idx = pl.pallas_call(
        _copy_mag_kernel,
        grid=(B, NCB),
        in_specs=[pl.BlockSpec((1, CB, H, W), lambda b, cb: (b, cb, 0, 0))],
        out_specs=[
            pl.BlockSpec((1, CB, H, W), lambda b, cb: (b, cb, 0, 0)),
            pl.BlockSpec((8, 128), lambda b, cb: (0, 0)),
        ],
        out_shape=[
            jax.ShapeDtypeStruct((B, C, H, W), input.dtype),
            jax.ShapeDtypeStruct((8, 128), jnp.int32),
        ],
        scratch_shapes=[pltpu.VMEM((NCB, CB), jnp.float32)],
    )(input)

    grid_spec = pltpu.PrefetchScalarGridSpec(
        num_scalar_prefetch=1,
        grid=(B, K),
        in_specs=[
            pl.BlockSpec((1, 1, H, W), lambda b, k, idx: (b, idx[b, k], 0, 0)),
        ],
        out_specs=pl.BlockSpec(
            (1, 1, H, W), lambda b, k, idx: (b, idx[b, k], 0, 0)),
    )
    y = pl.pallas_call(
        _zero_kernel,
        grid_spec=grid_spec,
        out_shape=jax.ShapeDtypeStruct((B, C, H, W), input.dtype),
        input_output_aliases={1: 0},
    )(idx, y1)
    return y


# speedup vs baseline: 1.4308x; 1.0784x over previous
import functools

import jax
import jax.numpy as jnp
from jax.experimental import pallas as pl
from jax.experimental.pallas import tpu as pltpu

B, C, H, W = 4, 192, 224, 224
CB = 32
NCB = C // CB
K = 8
BIG = 1 << 30


def _copy_mag_kernel(x_ref, y_ref, idx_ref, mag_ref):
    b = pl.program_id(0)
    cb = pl.program_id(1)
    x = x_ref[...]
    y_ref[...] = x
    mags = jnp.sum(x * x, axis=(-1, -2))
    mag_ref[pl.ds(cb, 1), :] = mags

    @pl.when(cb == NCB - 1)
    def _finish_batch():
        m = mag_ref[...]
        minval = jnp.min(m)
        r = jax.lax.broadcasted_iota(jnp.int32, (NCB, CB), 0)
        c = jax.lax.broadcasted_iota(jnp.int32, (NCB, CB), 1)
        flat = r * CB + c
        cand = jnp.where(m <= minval, flat, BIG)
        first = jnp.min(cand)
        lane = jax.lax.broadcasted_iota(jnp.int32, (1, 128), 1)
        row = jnp.full((1, 128), first, dtype=jnp.int32)
        cur = cand
        for k in range(K):
            mk = jnp.min(cur)
            sel = jnp.where(mk >= BIG, first, mk)
            row = jnp.where(lane == k, sel, row)
            cur = jnp.where(cand == mk, BIG, cur)
        idx_ref[pl.ds(b, 1), :] = row


def _zero_kernel(idx_ref, y_in_ref, y_out_ref):
    del idx_ref, y_in_ref
    y_out_ref[...] = jnp.zeros_like(y_out_ref)


@jax.jit
def kernel(input):
    y1, idx = pl.pallas_call(
        _copy_mag_kernel,
        grid=(B, NCB),
        in_specs=[pl.BlockSpec((1, CB, H, W), lambda b, cb: (b, cb, 0, 0))],
        out_specs=[
            pl.BlockSpec((1, CB, H, W), lambda b, cb: (b, cb, 0, 0)),
            pl.BlockSpec((8, 128), lambda b, cb: (0, 0)),
        ],
        out_shape=[
            jax.ShapeDtypeStruct((B, C, H, W), input.dtype),
            jax.ShapeDtypeStruct((8, 128), jnp.int32),
        ],
        scratch_shapes=[pltpu.VMEM((NCB, CB), jnp.float32)],
    )(input)

    grid_spec = pltpu.PrefetchScalarGridSpec(
        num_scalar_prefetch=1,
        grid=(B, K),
        in_specs=[
            pl.BlockSpec((1, 1, H, W), lambda b, k, idx: (b, idx[b, k], 0, 0)),
        ],
        out_specs=pl.BlockSpec(
            (1, 1, H, W), lambda b, k, idx: (b, idx[b, k], 0, 0)),
    )
    y = pl.pallas_call(
        _zero_kernel,
        grid_spec=grid_spec,
        out_shape=jax.ShapeDtypeStruct((B, C, H, W), input.dtype),
        input_output_aliases={1: 0},
    )(idx, y1)
    return y
